# Initial kernel scaffold; baseline (speedup 1.0000x reference)
import jax
import jax.numpy as jnp
from jax.experimental import pallas as pl


def kernel(vs, ve, ve_dead, Wq, Wk, Wv, Wm, bm, Wf, bf):
    raise NotImplementedError("write your pallas kernel here")


# speedup vs baseline: 1.4124x; 1.4124x over previous
import math

import jax
import jax.numpy as jnp
from jax import lax
from jax.experimental import pallas as pl

_R = 32


def _concentration_block(vs_ref, ve_ref, dead_ref, wq_ref, wk_ref, wv_ref,
                         wm_ref, bm_ref, wf_ref, bf_ref, outc_ref, outm_ref):
    R, N, H = ve_ref.shape
    f32 = jnp.float32

    vs = vs_ref[...]
    q = jnp.dot(vs, wq_ref[...], preferred_element_type=f32)

    ve = ve_ref[...]
    ve2 = ve.reshape(R * N, H)
    k2 = jnp.dot(ve2, wk_ref[...], preferred_element_type=f32)
    cf = lax.dot_general(q, k2, (((1,), (1,)), ((), ())),
                         preferred_element_type=f32)
    compat = jnp.concatenate(
        [cf[r:r + 1, r * N:(r + 1) * N] for r in range(R)], axis=0)
    compat = compat * (1.0 / math.sqrt(H))
    dead = dead_ref[...] != 0
    c2 = jnp.where(dead, -1e30, compat)
    mx = jnp.max(c2, axis=1, keepdims=True)
    e = jnp.where(dead, 0.0, jnp.exp(c2 - mx))
    s = jnp.sum(e, axis=1, keepdims=True)
    score = jnp.where(s > 0.0, e / s, 0.0)

    u = jnp.sum(score[:, :, None] * ve, axis=1)
    va = jnp.dot(u, wv_ref[...], preferred_element_type=f32)
    vm = (jnp.dot(vs, wm_ref[0:H, :], preferred_element_type=f32)
          + jnp.dot(va, wm_ref[H:2 * H, :], preferred_element_type=f32)
          + bm_ref[...])
    outm_ref[...] = jnp.maximum(vm, 0.0)

    nf = wf_ref.shape[0] // H - 1
    acc = jnp.dot(vs, wf_ref[0:H, :], preferred_element_type=f32) + bf_ref[...]
    cur = score
    iota = lax.broadcasted_iota(jnp.int32, (R, N), 1)
    for j in range(nf):
        mj = jnp.max(cur, axis=1, keepdims=True)
        eq = cur == mj
        first = jnp.min(jnp.where(eq, iota, N), axis=1, keepdims=True)
        oh = iota == first
        ohf = oh.astype(f32)
        g = jnp.sum(ve * ohf[:, :, None], axis=1)
        acc = acc + jnp.dot(g, wf_ref[(j + 1) * H:(j + 2) * H, :],
                            preferred_element_type=f32)
        cur = jnp.where(oh, -1.0, cur)
    outc_ref[...] = jnp.maximum(acc, 0.0)


def kernel(vs, ve, ve_dead, Wq, Wk, Wv, Wm, bm, Wf, bf):
    B, A, N, H = ve.shape
    BA = B * A
    R = _R
    vs2 = vs.reshape(BA, H)
    ve3 = ve.reshape(BA, N, H)
    dead2 = ve_dead.reshape(BA, N)
    nf1 = Wf.shape[0] // H

    outc, outm = pl.pallas_call(
        _concentration_block,
        grid=(BA // R,),
        in_specs=[
            pl.BlockSpec((R, H), lambda i: (i, 0)),
            pl.BlockSpec((R, N, H), lambda i: (i, 0, 0)),
            pl.BlockSpec((R, N), lambda i: (i, 0)),
            pl.BlockSpec((H, H), lambda i: (0, 0)),
            pl.BlockSpec((H, H), lambda i: (0, 0)),
            pl.BlockSpec((H, H), lambda i: (0, 0)),
            pl.BlockSpec((2 * H, H), lambda i: (0, 0)),
            pl.BlockSpec((1, H), lambda i: (0, 0)),
            pl.BlockSpec((nf1 * H, H), lambda i: (0, 0)),
            pl.BlockSpec((1, H), lambda i: (0, 0)),
        ],
        out_specs=[
            pl.BlockSpec((R, H), lambda i: (i, 0)),
            pl.BlockSpec((R, H), lambda i: (i, 0)),
        ],
        out_shape=[
            jax.ShapeDtypeStruct((BA, H), jnp.float32),
            jax.ShapeDtypeStruct((BA, H), jnp.float32),
        ],
    )(vs2, ve3, dead2, Wq, Wk, Wv, Wm, bm.reshape(1, H), Wf, bf.reshape(1, H))
    return outc.reshape(B, A, H), outm.reshape(B, A, H)


# speedup vs baseline: 2.6840x; 1.9003x over previous
import math

import jax
import jax.numpy as jnp
from jax import lax
from jax.experimental import pallas as pl
from jax.experimental.pallas import tpu as pltpu

_R = 32


def _concentration_block(vs_ref, ve_ref, dead_ref, wq_ref, wk_ref, wv_ref,
                         wm_ref, bm_ref, wf_ref, bf_ref, outc_ref, outm_ref,
                         w9_ref, g9_ref):
    R, N = dead_ref.shape
    H = vs_ref.shape[1]
    f32 = jnp.float32

    vs = vs_ref[...]
    q = jnp.dot(vs, wq_ref[...], preferred_element_type=f32)

    k2 = jnp.dot(ve_ref[...], wk_ref[...], preferred_element_type=f32)
    cf = lax.dot_general(q, k2, (((1,), (1,)), ((), ())),
                         preferred_element_type=f32)
    compat = jnp.concatenate(
        [cf[r:r + 1, r * N:(r + 1) * N] for r in range(R)], axis=0)
    compat = compat * (1.0 / math.sqrt(H))

    dead = dead_ref[...] != 0
    c2 = jnp.where(dead, -1e30, compat)
    mx = jnp.max(c2, axis=1, keepdims=True)
    e = jnp.where(dead, 0.0, jnp.exp(c2 - mx))
    s = jnp.sum(e, axis=1, keepdims=True)
    score = jnp.where(s > 0.0, e / s, 0.0)

    nf = wf_ref.shape[0] // H - 1
    w9_ref[:, 0, :] = score
    cur = score
    iota = lax.broadcasted_iota(jnp.int32, (R, N), 1)
    for j in range(nf):
        mj = jnp.max(cur, axis=1, keepdims=True)
        eq = cur == mj
        first = jnp.min(jnp.where(eq, iota, N), axis=1, keepdims=True)
        oh = iota == first
        w9_ref[:, j + 1, :] = oh.astype(f32)
        cur = jnp.where(oh, -1.0, cur)

    for r in range(R):
        g9_ref[:, r, :] = jnp.dot(w9_ref[r], ve_ref[r * N:(r + 1) * N, :],
                                  preferred_element_type=f32)

    u = g9_ref[0]
    va = jnp.dot(u, wv_ref[...], preferred_element_type=f32)
    vm = (jnp.dot(vs, wm_ref[0:H, :], preferred_element_type=f32)
          + jnp.dot(va, wm_ref[H:2 * H, :], preferred_element_type=f32)
          + bm_ref[...])
    outm_ref[...] = jnp.maximum(vm, 0.0)

    acc = jnp.dot(vs, wf_ref[0:H, :], preferred_element_type=f32) + bf_ref[...]
    for j in range(nf):
        acc = acc + jnp.dot(g9_ref[j + 1], wf_ref[(j + 1) * H:(j + 2) * H, :],
                            preferred_element_type=f32)
    outc_ref[...] = jnp.maximum(acc, 0.0)


def kernel(vs, ve, ve_dead, Wq, Wk, Wv, Wm, bm, Wf, bf):
    B, A, N, H = ve.shape
    BA = B * A
    R = _R
    vs2 = vs.reshape(BA, H)
    ve2 = ve.reshape(BA * N, H)
    dead2 = ve_dead.reshape(BA, N)
    nf1 = Wf.shape[0] // H

    outc, outm = pl.pallas_call(
        _concentration_block,
        grid=(BA // R,),
        in_specs=[
            pl.BlockSpec((R, H), lambda i: (i, 0)),
            pl.BlockSpec((R * N, H), lambda i: (i, 0)),
            pl.BlockSpec((R, N), lambda i: (i, 0)),
            pl.BlockSpec((H, H), lambda i: (0, 0)),
            pl.BlockSpec((H, H), lambda i: (0, 0)),
            pl.BlockSpec((H, H), lambda i: (0, 0)),
            pl.BlockSpec((2 * H, H), lambda i: (0, 0)),
            pl.BlockSpec((1, H), lambda i: (0, 0)),
            pl.BlockSpec((nf1 * H, H), lambda i: (0, 0)),
            pl.BlockSpec((1, H), lambda i: (0, 0)),
        ],
        out_specs=[
            pl.BlockSpec((R, H), lambda i: (i, 0)),
            pl.BlockSpec((R, H), lambda i: (i, 0)),
        ],
        out_shape=[
            jax.ShapeDtypeStruct((BA, H), jnp.float32),
            jax.ShapeDtypeStruct((BA, H), jnp.float32),
        ],
        scratch_shapes=[
            pltpu.VMEM((R, nf1, N), jnp.float32),
            pltpu.VMEM((nf1, R, H), jnp.float32),
        ],
    )(vs2, ve2, dead2, Wq, Wk, Wv, Wm, bm.reshape(1, H), Wf, bf.reshape(1, H))
    return outc.reshape(B, A, H), outm.reshape(B, A, H)


# speedup vs baseline: 3.6936x; 1.3762x over previous
import math

import jax
import jax.numpy as jnp
from jax import lax
from jax.experimental import pallas as pl
from jax.experimental.pallas import tpu as pltpu

_R = 64


def _concentration_block(vs_ref, ve_ref, dead_ref, wq_ref, wk_ref, wv_ref,
                         wm_ref, bm_ref, wf_ref, bf_ref, outc_ref, outm_ref,
                         w9_ref, g9_ref):
    R, N = dead_ref.shape
    H = vs_ref.shape[1]
    f32 = jnp.float32

    vs = vs_ref[...]
    q = jnp.dot(vs, wq_ref[...], preferred_element_type=f32)

    k2 = jnp.dot(ve_ref[...], wk_ref[...], preferred_element_type=f32)
    cf = lax.dot_general(q, k2, (((1,), (1,)), ((), ())),
                         preferred_element_type=f32)
    compat = jnp.concatenate(
        [cf[r:r + 1, r * N:(r + 1) * N] for r in range(R)], axis=0)
    compat = compat * (1.0 / math.sqrt(H))

    dead = dead_ref[...] != 0
    c2 = jnp.where(dead, -1e30, compat)
    mx = jnp.max(c2, axis=1, keepdims=True)
    e = jnp.where(dead, 0.0, jnp.exp(c2 - mx))
    s = jnp.sum(e, axis=1, keepdims=True)
    score = jnp.where(s > 0.0, e / s, 0.0)

    nf = wf_ref.shape[0] // H - 1
    w9_ref[:, 0, :] = score
    cur = score
    iota = lax.broadcasted_iota(jnp.int32, (R, N), 1)
    for j in range(nf):
        mj = jnp.max(cur, axis=1, keepdims=True)
        eq = cur == mj
        first = jnp.min(jnp.where(eq, iota, N), axis=1, keepdims=True)
        oh = iota == first
        w9_ref[:, j + 1, :] = oh.astype(f32)
        cur = jnp.where(oh, -1.0, cur)

    for r in range(R):
        g9_ref[:, r, :] = jnp.dot(w9_ref[r], ve_ref[r * N:(r + 1) * N, :],
                                  preferred_element_type=f32)

    u = g9_ref[0]
    va = jnp.dot(u, wv_ref[...], preferred_element_type=f32)
    vm = (jnp.dot(vs, wm_ref[0:H, :], preferred_element_type=f32)
          + jnp.dot(va, wm_ref[H:2 * H, :], preferred_element_type=f32)
          + bm_ref[...])
    outm_ref[...] = jnp.maximum(vm, 0.0)

    acc = jnp.dot(vs, wf_ref[0:H, :], preferred_element_type=f32) + bf_ref[...]
    for j in range(nf):
        acc = acc + jnp.dot(g9_ref[j + 1], wf_ref[(j + 1) * H:(j + 2) * H, :],
                            preferred_element_type=f32)
    outc_ref[...] = jnp.maximum(acc, 0.0)


def kernel(vs, ve, ve_dead, Wq, Wk, Wv, Wm, bm, Wf, bf):
    B, A, N, H = ve.shape
    BA = B * A
    R = _R
    vs2 = vs.reshape(BA, H)
    ve2 = ve.reshape(BA * N, H)
    dead2 = ve_dead.reshape(BA, N)
    nf1 = Wf.shape[0] // H

    outc, outm = pl.pallas_call(
        _concentration_block,
        grid=(BA // R,),
        in_specs=[
            pl.BlockSpec((R, H), lambda i: (i, 0)),
            pl.BlockSpec((R * N, H), lambda i: (i, 0)),
            pl.BlockSpec((R, N), lambda i: (i, 0)),
            pl.BlockSpec((H, H), lambda i: (0, 0)),
            pl.BlockSpec((H, H), lambda i: (0, 0)),
            pl.BlockSpec((H, H), lambda i: (0, 0)),
            pl.BlockSpec((2 * H, H), lambda i: (0, 0)),
            pl.BlockSpec((1, H), lambda i: (0, 0)),
            pl.BlockSpec((nf1 * H, H), lambda i: (0, 0)),
            pl.BlockSpec((1, H), lambda i: (0, 0)),
        ],
        out_specs=[
            pl.BlockSpec((R, H), lambda i: (i, 0)),
            pl.BlockSpec((R, H), lambda i: (i, 0)),
        ],
        out_shape=[
            jax.ShapeDtypeStruct((BA, H), jnp.float32),
            jax.ShapeDtypeStruct((BA, H), jnp.float32),
        ],
        scratch_shapes=[
            pltpu.VMEM((R, nf1, N), jnp.float32),
            pltpu.VMEM((nf1, R, H), jnp.float32),
        ],
    )(vs2, ve2, dead2, Wq, Wk, Wv, Wm, bm.reshape(1, H), Wf, bf.reshape(1, H))
    return outc.reshape(B, A, H), outm.reshape(B, A, H)


# speedup vs baseline: 4.2876x; 1.1608x over previous
import math

import jax
import jax.numpy as jnp
from jax import lax
from jax.experimental import pallas as pl
from jax.experimental.pallas import tpu as pltpu

_R = 128


def _concentration_block(vs_ref, ve_ref, dead_ref, wq_ref, wk_ref, wv_ref,
                         wm_ref, bm_ref, wf_ref, bf_ref, outc_ref, outm_ref,
                         w9_ref, g9_ref):
    R, N = dead_ref.shape
    H = vs_ref.shape[1]
    f32 = jnp.float32

    vs = vs_ref[...]
    q = jnp.dot(vs, wq_ref[...], preferred_element_type=f32)

    k2 = jnp.dot(ve_ref[...], wk_ref[...], preferred_element_type=f32)
    G = 8
    parts = []
    for s0 in range(0, R, G):
        cf = lax.dot_general(q[s0:s0 + G], k2[s0 * N:(s0 + G) * N],
                             (((1,), (1,)), ((), ())),
                             preferred_element_type=f32)
        parts.extend(cf[r:r + 1, r * N:(r + 1) * N] for r in range(G))
    compat = jnp.concatenate(parts, axis=0)
    compat = compat * (1.0 / math.sqrt(H))

    dead = dead_ref[...] != 0
    c2 = jnp.where(dead, -1e30, compat)
    mx = jnp.max(c2, axis=1, keepdims=True)
    e = jnp.where(dead, 0.0, jnp.exp(c2 - mx))
    s = jnp.sum(e, axis=1, keepdims=True)
    score = jnp.where(s > 0.0, e / s, 0.0)

    nf = wf_ref.shape[0] // H - 1
    w9_ref[:, 0, :] = score
    cur = score
    iota = lax.broadcasted_iota(jnp.int32, (R, N), 1)
    for j in range(nf):
        mj = jnp.max(cur, axis=1, keepdims=True)
        eq = cur == mj
        first = jnp.min(jnp.where(eq, iota, N), axis=1, keepdims=True)
        oh = iota == first
        w9_ref[:, j + 1, :] = oh.astype(f32)
        cur = jnp.where(oh, -1.0, cur)

    for r in range(R):
        g9_ref[:, r, :] = jnp.dot(w9_ref[r], ve_ref[r * N:(r + 1) * N, :],
                                  preferred_element_type=f32)

    u = g9_ref[0]
    va = jnp.dot(u, wv_ref[...], preferred_element_type=f32)
    vm = (jnp.dot(vs, wm_ref[0:H, :], preferred_element_type=f32)
          + jnp.dot(va, wm_ref[H:2 * H, :], preferred_element_type=f32)
          + bm_ref[...])
    outm_ref[...] = jnp.maximum(vm, 0.0)

    acc = jnp.dot(vs, wf_ref[0:H, :], preferred_element_type=f32) + bf_ref[...]
    for j in range(nf):
        acc = acc + jnp.dot(g9_ref[j + 1], wf_ref[(j + 1) * H:(j + 2) * H, :],
                            preferred_element_type=f32)
    outc_ref[...] = jnp.maximum(acc, 0.0)


def kernel(vs, ve, ve_dead, Wq, Wk, Wv, Wm, bm, Wf, bf):
    B, A, N, H = ve.shape
    BA = B * A
    R = _R
    vs2 = vs.reshape(BA, H)
    ve2 = ve.reshape(BA * N, H)
    dead2 = ve_dead.reshape(BA, N)
    nf1 = Wf.shape[0] // H

    outc, outm = pl.pallas_call(
        _concentration_block,
        grid=(BA // R,),
        in_specs=[
            pl.BlockSpec((R, H), lambda i: (i, 0)),
            pl.BlockSpec((R * N, H), lambda i: (i, 0)),
            pl.BlockSpec((R, N), lambda i: (i, 0)),
            pl.BlockSpec((H, H), lambda i: (0, 0)),
            pl.BlockSpec((H, H), lambda i: (0, 0)),
            pl.BlockSpec((H, H), lambda i: (0, 0)),
            pl.BlockSpec((2 * H, H), lambda i: (0, 0)),
            pl.BlockSpec((1, H), lambda i: (0, 0)),
            pl.BlockSpec((nf1 * H, H), lambda i: (0, 0)),
            pl.BlockSpec((1, H), lambda i: (0, 0)),
        ],
        out_specs=[
            pl.BlockSpec((R, H), lambda i: (i, 0)),
            pl.BlockSpec((R, H), lambda i: (i, 0)),
        ],
        out_shape=[
            jax.ShapeDtypeStruct((BA, H), jnp.float32),
            jax.ShapeDtypeStruct((BA, H), jnp.float32),
        ],
        scratch_shapes=[
            pltpu.VMEM((R, nf1, N), jnp.float32),
            pltpu.VMEM((nf1, R, H), jnp.float32),
        ],
    )(vs2, ve2, dead2, Wq, Wk, Wv, Wm, bm.reshape(1, H), Wf, bf.reshape(1, H))
    return outc.reshape(B, A, H), outm.reshape(B, A, H)
